# baseline (speedup 1.0000x reference)
import functools

import jax
import jax.numpy as jnp
from jax import lax
from jax.experimental import pallas as pl
from jax.experimental.pallas import tpu as pltpu
from jax.experimental.pallas import tpu_sc as plsc

_F32 = jnp.float32

_NC = 2
_NS = 16
_LANES = 16


@functools.partial(jax.jit, static_argnames=("interpret",))
def _sc_aggregate(h, ep, edge_index, *, interpret=False):
    N, D = h.shape
    E = ep.shape[0]
    NW = _NC * _NS
    EPC = E // _NC
    EPW = E // NW
    C = 40
    G = EPW // C
    RZ = C
    NP = ((N + RZ * _NS - 1) // (RZ * _NS)) * (RZ * _NS)
    RPT = NP // _NS
    NB = RPT // RZ
    assert EPW % C == 0 and G % 2 == 0 and G >= 4
    assert RPT % RZ == 0 and D % _LANES == 0

    mesh = plsc.VectorSubcoreMesh(core_axis_name="c", subcore_axis_name="s")

    @functools.partial(
        pl.kernel,
        out_type=jax.ShapeDtypeStruct((_NC, NP, D), _F32),
        mesh=mesh,
        interpret=interpret,
        compiler_params=pltpu.CompilerParams(use_tc_tiling_on_sc=False),
        scratch_types=[
            pltpu.VMEM((EPW,), jnp.int32),
            pltpu.VMEM((3, C), jnp.int32),
            pltpu.VMEM((3, C, D), _F32),
            pltpu.VMEM((3, C, D), _F32),
            pltpu.VMEM_SHARED((NP, D), _F32),
            pltpu.SemaphoreType.DMA, pltpu.SemaphoreType.DMA,
            pltpu.SemaphoreType.DMA,
            pltpu.SemaphoreType.DMA, pltpu.SemaphoreType.DMA,
            pltpu.SemaphoreType.DMA,
            pltpu.SemaphoreType.DMA, pltpu.SemaphoreType.DMA,
            pltpu.SemaphoreType.DMA,
            pltpu.SemaphoreType.DMA, pltpu.SemaphoreType.DMA,
            pltpu.SemaphoreType.DMA,
        ],
    )
    def agg_kernel(h_hbm, ep_hbm, ei_hbm, out_hbm,
                   src_full, dst_v, hrow_v, msg_v, agg_sh,
                   sd0, sd1, sd2, se0, se1, se2, sg0, sg1, sg2,
                   ss0, ss1, ss2):
        sem_d = (sd0, sd1, sd2)
        sem_e = (se0, se1, se2)
        sem_g = (sg0, sg1, sg2)
        sem_s = (ss0, ss1, ss2)
        cid = lax.axis_index("c")
        sid = lax.axis_index("s")

        def zero_row(i, carry):
            for j in range(D // _LANES):
                msg_v[0, i, pl.ds(j * _LANES, _LANES)] = jnp.zeros(
                    (_LANES,), _F32)
            return carry
        lax.fori_loop(0, RZ, zero_row, 0)
        for b in range(NB):
            pltpu.async_copy(msg_v.at[0],
                             agg_sh.at[pl.ds(sid * RPT + b * RZ, RZ)],
                             sem_e[b % 3])
        for b in range(NB):
            pltpu.make_async_copy(
                msg_v.at[0], agg_sh.at[pl.ds(sid * RPT + b * RZ, RZ)],
                sem_e[b % 3]).wait()
        plsc.subcore_barrier()

        wbase = cid * EPC + sid * EPW
        pltpu.sync_copy(ei_hbm.at[0, pl.ds(wbase, EPW)], src_full)

        def issue_loads(g, b):
            eb = pl.multiple_of(wbase + g * C, 8)
            pltpu.async_copy(ei_hbm.at[1, pl.ds(eb, C)], dst_v.at[b],
                             sem_d[b])
            pltpu.async_copy(ep_hbm.at[pl.ds(eb, C)], msg_v.at[b],
                             sem_e[b])

        def issue_gather(g, b):
            idx = src_full.at[pl.ds(pl.multiple_of(g * C, 8), C)]
            pltpu.async_copy(h_hbm.at[idx], hrow_v.at[b], sem_g[b])

        def issue_scatter(b):
            pltpu.async_copy(msg_v.at[b], agg_sh.at[dst_v.at[b]], sem_s[b],
                             add=True)

        def wait_loads(g, b):
            eb = pl.multiple_of(wbase + g * C, 8)
            pltpu.make_async_copy(ep_hbm.at[pl.ds(eb, C)], msg_v.at[b],
                                  sem_e[b]).wait()

        def wait_dst(g, b):
            eb = pl.multiple_of(wbase + g * C, 8)
            pltpu.make_async_copy(ei_hbm.at[1, pl.ds(eb, C)], dst_v.at[b],
                                  sem_d[b]).wait()

        def wait_gather(g, b):
            idx = src_full.at[pl.ds(pl.multiple_of(g * C, 8), C)]
            pltpu.make_async_copy(h_hbm.at[idx], hrow_v.at[b],
                                  sem_g[b]).wait()

        def wait_scatter(b):
            pltpu.make_async_copy(msg_v.at[b], agg_sh.at[dst_v.at[b]],
                                  sem_s[b]).wait()

        def compute(b):
            @plsc.parallel_loop(0, C, unroll=8)
            def _(r):
                for j in range(D // _LANES):
                    sl = pl.ds(j * _LANES, _LANES)
                    msg_v[b, r, sl] = jnp.maximum(
                        hrow_v[b, r, sl] + msg_v[b, r, sl], 0.0)

        def consume(g, b):
            wait_loads(g, b)
            wait_gather(g, b)
            compute(b)
            wait_dst(g, b)
            issue_scatter(b)

        GL = G - 1
        NO = (GL - 3) // 3
        assert G % 2 == 0 and (GL - 3) % 3 == 0

        for g0 in (0, 1, 2):
            issue_loads(g0, g0)
            issue_gather(g0, g0)
        consume(0, 0)
        consume(1, 1)
        wait_scatter(0)
        issue_loads(3, 0)
        issue_gather(3, 0)
        consume(2, 2)

        def outer(o, carry):
            for b in (0, 1, 2):
                g = 3 + 3 * o + b
                pb = (b + 1) % 3
                wait_scatter(pb)
                issue_loads(g + 1, pb)
                issue_gather(g + 1, pb)
                consume(g, b)
            return carry
        lax.fori_loop(0, NO, outer, 0)

        consume(GL, 0)
        wait_scatter(0)
        wait_scatter(1)
        wait_scatter(2)
        plsc.subcore_barrier()

        def wb_rows(k):
            return pl.ds(sid * RPT + k * RZ, RZ)

        def wb_load(k):
            pltpu.async_copy(agg_sh.at[wb_rows(k)], msg_v.at[k % 3],
                             sem_g[k % 3])

        def wb_load_wait(k):
            pltpu.make_async_copy(agg_sh.at[wb_rows(k)], msg_v.at[k % 3],
                                  sem_g[k % 3]).wait()

        def wb_store(k):
            pltpu.async_copy(msg_v.at[k % 3], out_hbm.at[cid, wb_rows(k)],
                             sem_s[k % 3])

        def wb_store_wait(k):
            pltpu.make_async_copy(msg_v.at[k % 3],
                                  out_hbm.at[cid, wb_rows(k)],
                                  sem_s[k % 3]).wait()

        wb_load(0)
        for k in range(NB):
            if k + 1 < NB:
                if k - 2 >= 0:
                    wb_store_wait(k - 2)
                wb_load(k + 1)
            wb_load_wait(k)
            wb_store(k)
        for k in range(max(0, NB - 3), NB):
            wb_store_wait(k)

    return agg_kernel(h, ep, edge_index)


def _edge_proj1(edge_attr, We0, be0):
    E, DE = edge_attr.shape
    BE = 2560
    grid = (E // BE,)
    d0 = We0.shape[1]

    def body(ea, w0, b0, o0):
        o0[...] = jnp.dot(ea[...], w0[...],
                          preferred_element_type=_F32) + b0[...]

    def full(w):
        return pl.BlockSpec(w.shape, lambda i: (0, 0))

    return pl.pallas_call(
        body,
        grid=grid,
        in_specs=[pl.BlockSpec((BE, DE), lambda i: (i, 0)),
                  full(We0), full(be0)],
        out_specs=pl.BlockSpec((BE, d0), lambda i: (i, 0)),
        out_shape=jax.ShapeDtypeStruct((E, d0), _F32),
    )(edge_attr, We0, be0)


def _edge_proj2(edge_attr, We1, be1, We2, be2):
    E, DE = edge_attr.shape
    BE = 2560
    grid = (E // BE,)

    def body(ea, w1, b1, w2, b2, o1, o2):
        a = ea[...]
        o1[...] = jnp.dot(a, w1[...], preferred_element_type=_F32) + b1[...]
        o2[...] = jnp.dot(a, w2[...], preferred_element_type=_F32) + b2[...]

    def full(w):
        return pl.BlockSpec(w.shape, lambda i: (0, 0))

    d1, d2 = We1.shape[1], We2.shape[1]
    return pl.pallas_call(
        body,
        grid=grid,
        in_specs=[
            pl.BlockSpec((BE, DE), lambda i: (i, 0)),
            full(We1), full(be1), full(We2), full(be2),
        ],
        out_specs=[
            pl.BlockSpec((BE, d1), lambda i: (i, 0)),
            pl.BlockSpec((BE, d2), lambda i: (i, 0)),
        ],
        out_shape=[
            jax.ShapeDtypeStruct((E, d1), _F32),
            jax.ShapeDtypeStruct((E, d2), _F32),
        ],
    )(edge_attr, We1, be1, We2, be2)


def _node_mlp(h, agg, W1, b1, W2, b2):
    N, D = h.shape
    H = W1.shape[1]
    BN = 2000
    grid = (N // BN,)

    def body(h_ref, a0_ref, a1_ref, w1, bb1, w2, bb2, o):
        z = h_ref[...] + a0_ref[0] + a1_ref[0]
        t = jnp.maximum(
            jnp.dot(z, w1[...], preferred_element_type=_F32) + bb1[...], 0.0)
        o[...] = jnp.dot(t, w2[...], preferred_element_type=_F32) + bb2[...]

    def full(w):
        return pl.BlockSpec(w.shape, lambda i: (0, 0))

    return pl.pallas_call(
        body,
        grid=grid,
        in_specs=[
            pl.BlockSpec((BN, D), lambda i: (i, 0)),
            pl.BlockSpec((1, BN, D), lambda i: (0, i, 0)),
            pl.BlockSpec((1, BN, D), lambda i: (1, i, 0)),
            full(W1), full(b1), full(W2), full(b2),
        ],
        out_specs=pl.BlockSpec((BN, H), lambda i: (i, 0)),
        out_shape=jax.ShapeDtypeStruct((N, H), _F32),
    )(h, agg, agg, W1, b1, W2, b2)


def _node_mlp_final(h, agg, W1, b1, W2, b2, gamma, beta, prompt,
                    Wp1, bp1, Wp2, bp2):
    N, D = h.shape
    BN = 2000
    grid = (N // BN,)

    def body(h_ref, a0_ref, a1_ref, w1, bb1, w2, bb2, g_ref, be_ref, p_ref,
             wp1, bbp1, wp2, bbp2, o):
        z = h_ref[...] + a0_ref[0] + a1_ref[0]
        t = jnp.maximum(
            jnp.dot(z, w1[...], preferred_element_type=_F32) + bb1[...], 0.0)
        hh = jnp.dot(t, w2[...], preferred_element_type=_F32) + bb2[...]
        mu = jnp.mean(hh, axis=-1, keepdims=True)
        dmu = hh - mu
        var = jnp.mean(dmu * dmu, axis=-1, keepdims=True)
        hn = dmu * lax.rsqrt(var + 1e-5) * g_ref[...] + be_ref[...]
        g = p_ref[...] + hn
        u = jnp.maximum(
            jnp.dot(g, wp1[...], preferred_element_type=_F32) + bbp1[...], 0.0)
        o[...] = jnp.maximum(
            jnp.dot(u, wp2[...], preferred_element_type=_F32) + bbp2[...], 0.0)

    def full(w):
        return pl.BlockSpec(w.shape, lambda i: (0, 0))

    return pl.pallas_call(
        body,
        grid=grid,
        in_specs=[
            pl.BlockSpec((BN, D), lambda i: (i, 0)),
            pl.BlockSpec((1, BN, D), lambda i: (0, i, 0)),
            pl.BlockSpec((1, BN, D), lambda i: (1, i, 0)),
            full(W1), full(b1), full(W2), full(b2),
            full(gamma), full(beta), full(prompt),
            full(Wp1), full(bp1), full(Wp2), full(bp2),
        ],
        out_specs=pl.BlockSpec((BN, 1), lambda i: (i, 0)),
        out_shape=jax.ShapeDtypeStruct((N, 1), _F32),
    )(h, agg, agg, W1, b1, W2, b2, gamma, beta, prompt, Wp1, bp1, Wp2, bp2)


def kernel(x, edge_index, edge_attr, We0, be0, W10, b10, W20, b20,
           We1, be1, W11, b11, W21, b21, We2, be2, W12, b12, W22, b22,
           gamma, beta, prompt, Wp1, bp1, Wp2, bp2):
    r2 = lambda v: v.reshape(1, -1)
    ep0 = _edge_proj1(edge_attr, We0, r2(be0))
    ep1, ep2 = _edge_proj2(edge_attr, We1, r2(be1), We2, r2(be2))

    h = x
    agg = _sc_aggregate(h, ep0, edge_index)
    h = _node_mlp(h, agg, W10, r2(b10), W20, r2(b20))

    agg = _sc_aggregate(h, ep1, edge_index)
    h = _node_mlp(h, agg, W11, r2(b11), W21, r2(b21))

    agg = _sc_aggregate(h, ep2, edge_index)
    return _node_mlp_final(h, agg, W12, r2(b12), W22, r2(b22),
                           r2(gamma), r2(beta), prompt,
                           Wp1, r2(bp1), Wp2, r2(bp2))


# speedup vs baseline: 1.2510x; 1.2510x over previous
import functools

import jax
import jax.numpy as jnp
from jax import lax
from jax.experimental import pallas as pl
from jax.experimental.pallas import tpu as pltpu
from jax.experimental.pallas import tpu_sc as plsc

_F32 = jnp.float32

_NC = 2
_NS = 16
_LANES = 16


@functools.partial(jax.jit, static_argnames=("interpret",))
def _sc_aggregate(h, ep, edge_index, *, interpret=False):
    N, D = h.shape
    E = ep.shape[0]
    NW = _NC * _NS
    EPC = E // _NC
    EPW = E // NW
    C = 40
    G = EPW // C
    RZ = C
    NP = ((N + RZ * _NS - 1) // (RZ * _NS)) * (RZ * _NS)
    RPT = NP // _NS
    NB = RPT // RZ
    assert EPW % C == 0 and G % 2 == 0 and G >= 4
    assert RPT % RZ == 0 and D % _LANES == 0

    mesh = plsc.VectorSubcoreMesh(core_axis_name="c", subcore_axis_name="s")

    @functools.partial(
        pl.kernel,
        out_type=jax.ShapeDtypeStruct((_NC, NP, D), _F32),
        mesh=mesh,
        interpret=interpret,
        compiler_params=pltpu.CompilerParams(use_tc_tiling_on_sc=False),
        scratch_types=[
            pltpu.VMEM((EPW,), jnp.int32),
            pltpu.VMEM((3, C), jnp.int32),
            pltpu.VMEM((3, C, D), _F32),
            pltpu.VMEM((3, C, D), _F32),
            pltpu.VMEM_SHARED((NP, D), _F32),
            pltpu.SemaphoreType.DMA, pltpu.SemaphoreType.DMA,
            pltpu.SemaphoreType.DMA,
            pltpu.SemaphoreType.DMA, pltpu.SemaphoreType.DMA,
            pltpu.SemaphoreType.DMA,
            pltpu.SemaphoreType.DMA, pltpu.SemaphoreType.DMA,
            pltpu.SemaphoreType.DMA,
            pltpu.SemaphoreType.DMA, pltpu.SemaphoreType.DMA,
            pltpu.SemaphoreType.DMA,
        ],
    )
    def agg_kernel(h_hbm, ep_hbm, ei_hbm, out_hbm,
                   src_full, dst_v, hrow_v, msg_v, agg_sh,
                   sd0, sd1, sd2, se0, se1, se2, sg0, sg1, sg2,
                   ss0, ss1, ss2):
        sem_d = (sd0, sd1, sd2)
        sem_e = (se0, se1, se2)
        sem_g = (sg0, sg1, sg2)
        sem_s = (ss0, ss1, ss2)
        cid = lax.axis_index("c")
        sid = lax.axis_index("s")

        def zero_row(i, carry):
            for j in range(D // _LANES):
                msg_v[0, i, pl.ds(j * _LANES, _LANES)] = jnp.zeros(
                    (_LANES,), _F32)
            return carry
        lax.fori_loop(0, RZ, zero_row, 0)
        for b in range(NB):
            pltpu.async_copy(msg_v.at[0],
                             agg_sh.at[pl.ds(sid * RPT + b * RZ, RZ)],
                             sem_e[b % 3])
        for b in range(NB):
            pltpu.make_async_copy(
                msg_v.at[0], agg_sh.at[pl.ds(sid * RPT + b * RZ, RZ)],
                sem_e[b % 3]).wait()
        plsc.subcore_barrier()

        wbase = cid * EPC + sid * EPW
        pltpu.sync_copy(ei_hbm.at[0, pl.ds(wbase, EPW)], src_full)

        def issue_loads(g, b):
            eb = pl.multiple_of(wbase + g * C, 8)
            pltpu.async_copy(ei_hbm.at[1, pl.ds(eb, C)], dst_v.at[b],
                             sem_d[b])
            pltpu.async_copy(ep_hbm.at[pl.ds(eb, C)], msg_v.at[b],
                             sem_e[b])

        def issue_gather(g, b):
            idx = src_full.at[pl.ds(pl.multiple_of(g * C, 8), C)]
            pltpu.async_copy(h_hbm.at[idx], hrow_v.at[b], sem_g[b])

        def issue_scatter(b):
            pltpu.async_copy(msg_v.at[b], agg_sh.at[dst_v.at[b]], sem_s[b],
                             add=True)

        def wait_loads(g, b):
            eb = pl.multiple_of(wbase + g * C, 8)
            pltpu.make_async_copy(ep_hbm.at[pl.ds(eb, C)], msg_v.at[b],
                                  sem_e[b]).wait()

        def wait_dst(g, b):
            eb = pl.multiple_of(wbase + g * C, 8)
            pltpu.make_async_copy(ei_hbm.at[1, pl.ds(eb, C)], dst_v.at[b],
                                  sem_d[b]).wait()

        def wait_gather(g, b):
            idx = src_full.at[pl.ds(pl.multiple_of(g * C, 8), C)]
            pltpu.make_async_copy(h_hbm.at[idx], hrow_v.at[b],
                                  sem_g[b]).wait()

        def wait_scatter(b):
            pltpu.make_async_copy(msg_v.at[b], agg_sh.at[dst_v.at[b]],
                                  sem_s[b]).wait()

        def compute(b):
            @plsc.parallel_loop(0, C, unroll=8)
            def _(r):
                for j in range(D // _LANES):
                    sl = pl.ds(j * _LANES, _LANES)
                    msg_v[b, r, sl] = jnp.maximum(
                        hrow_v[b, r, sl] + msg_v[b, r, sl], 0.0)

        def consume(g, b):
            wait_loads(g, b)
            wait_gather(g, b)
            compute(b)
            wait_dst(g, b)
            issue_scatter(b)

        GL = G - 1
        NO = (GL - 3) // 3
        assert G % 2 == 0 and (GL - 3) % 3 == 0

        for g0 in (0, 1, 2):
            issue_loads(g0, g0)
            issue_gather(g0, g0)
        consume(0, 0)
        consume(1, 1)
        wait_scatter(0)
        issue_loads(3, 0)
        issue_gather(3, 0)
        consume(2, 2)

        def outer(o, carry):
            for b in (0, 1, 2):
                g = 3 + 3 * o + b
                pb = (b + 1) % 3
                wait_scatter(pb)
                issue_loads(g + 1, pb)
                issue_gather(g + 1, pb)
                consume(g, b)
            return carry
        lax.fori_loop(0, NO, outer, 0)

        consume(GL, 0)
        wait_scatter(0)
        wait_scatter(1)
        wait_scatter(2)
        plsc.subcore_barrier()

        def wb_rows(k):
            return pl.ds(sid * RPT + k * RZ, RZ)

        def wb_load(k):
            pltpu.async_copy(agg_sh.at[wb_rows(k)], msg_v.at[k % 3],
                             sem_g[k % 3])

        def wb_load_wait(k):
            pltpu.make_async_copy(agg_sh.at[wb_rows(k)], msg_v.at[k % 3],
                                  sem_g[k % 3]).wait()

        def wb_store(k):
            pltpu.async_copy(msg_v.at[k % 3], out_hbm.at[cid, wb_rows(k)],
                             sem_s[k % 3])

        def wb_store_wait(k):
            pltpu.make_async_copy(msg_v.at[k % 3],
                                  out_hbm.at[cid, wb_rows(k)],
                                  sem_s[k % 3]).wait()

        wb_load(0)
        for k in range(NB):
            if k + 1 < NB:
                if k - 2 >= 0:
                    wb_store_wait(k - 2)
                wb_load(k + 1)
            wb_load_wait(k)
            wb_store(k)
        for k in range(max(0, NB - 3), NB):
            wb_store_wait(k)

    return agg_kernel(h, ep, edge_index)


def _edge_proj(ea_t, We0, be0, We1, be1, We2, be2):
    DE, E = ea_t.shape
    BE = 2560
    grid = (E // BE,)
    dn = (((0,), (0,)), ((), ()))

    def body(ea, w0, b0, w1, b1, w2, b2, o0, o1, o2):
        a = ea[...]
        o0[...] = lax.dot_general(a, w0[...], dn,
                                  preferred_element_type=_F32) + b0[...]
        o1[...] = lax.dot_general(a, w1[...], dn,
                                  preferred_element_type=_F32) + b1[...]
        o2[...] = lax.dot_general(a, w2[...], dn,
                                  preferred_element_type=_F32) + b2[...]

    def full(w):
        return pl.BlockSpec(w.shape, lambda i: (0, 0))

    d0, d1, d2 = We0.shape[1], We1.shape[1], We2.shape[1]
    return pl.pallas_call(
        body,
        grid=grid,
        in_specs=[
            pl.BlockSpec((DE, BE), lambda i: (0, i)),
            full(We0), full(be0), full(We1), full(be1), full(We2), full(be2),
        ],
        out_specs=[
            pl.BlockSpec((BE, d0), lambda i: (i, 0)),
            pl.BlockSpec((BE, d1), lambda i: (i, 0)),
            pl.BlockSpec((BE, d2), lambda i: (i, 0)),
        ],
        out_shape=[
            jax.ShapeDtypeStruct((E, d0), _F32),
            jax.ShapeDtypeStruct((E, d1), _F32),
            jax.ShapeDtypeStruct((E, d2), _F32),
        ],
    )(ea_t, We0, be0, We1, be1, We2, be2)


def _node_mlp(h, agg, W1, b1, W2, b2):
    N, D = h.shape
    H = W1.shape[1]
    BN = 2000
    grid = (N // BN,)

    def body(h_ref, a0_ref, a1_ref, w1, bb1, w2, bb2, o):
        z = h_ref[...] + a0_ref[0] + a1_ref[0]
        t = jnp.maximum(
            jnp.dot(z, w1[...], preferred_element_type=_F32) + bb1[...], 0.0)
        o[...] = jnp.dot(t, w2[...], preferred_element_type=_F32) + bb2[...]

    def full(w):
        return pl.BlockSpec(w.shape, lambda i: (0, 0))

    return pl.pallas_call(
        body,
        grid=grid,
        in_specs=[
            pl.BlockSpec((BN, D), lambda i: (i, 0)),
            pl.BlockSpec((1, BN, D), lambda i: (0, i, 0)),
            pl.BlockSpec((1, BN, D), lambda i: (1, i, 0)),
            full(W1), full(b1), full(W2), full(b2),
        ],
        out_specs=pl.BlockSpec((BN, H), lambda i: (i, 0)),
        out_shape=jax.ShapeDtypeStruct((N, H), _F32),
    )(h, agg, agg, W1, b1, W2, b2)


def _node_mlp_final(h, agg, W1, b1, W2, b2, gamma, beta, prompt,
                    Wp1, bp1, Wp2, bp2):
    N, D = h.shape
    BN = 2000
    grid = (N // BN,)

    def body(h_ref, a0_ref, a1_ref, w1, bb1, w2, bb2, g_ref, be_ref, p_ref,
             wp1, bbp1, wp2, bbp2, o):
        z = h_ref[...] + a0_ref[0] + a1_ref[0]
        t = jnp.maximum(
            jnp.dot(z, w1[...], preferred_element_type=_F32) + bb1[...], 0.0)
        hh = jnp.dot(t, w2[...], preferred_element_type=_F32) + bb2[...]
        mu = jnp.mean(hh, axis=-1, keepdims=True)
        dmu = hh - mu
        var = jnp.mean(dmu * dmu, axis=-1, keepdims=True)
        hn = dmu * lax.rsqrt(var + 1e-5) * g_ref[...] + be_ref[...]
        g = p_ref[...] + hn
        u = jnp.maximum(
            jnp.dot(g, wp1[...], preferred_element_type=_F32) + bbp1[...], 0.0)
        o[...] = jnp.maximum(
            jnp.dot(u, wp2[...], preferred_element_type=_F32) + bbp2[...], 0.0)

    def full(w):
        return pl.BlockSpec(w.shape, lambda i: (0, 0))

    return pl.pallas_call(
        body,
        grid=grid,
        in_specs=[
            pl.BlockSpec((BN, D), lambda i: (i, 0)),
            pl.BlockSpec((1, BN, D), lambda i: (0, i, 0)),
            pl.BlockSpec((1, BN, D), lambda i: (1, i, 0)),
            full(W1), full(b1), full(W2), full(b2),
            full(gamma), full(beta), full(prompt),
            full(Wp1), full(bp1), full(Wp2), full(bp2),
        ],
        out_specs=pl.BlockSpec((BN, 1), lambda i: (i, 0)),
        out_shape=jax.ShapeDtypeStruct((N, 1), _F32),
    )(h, agg, agg, W1, b1, W2, b2, gamma, beta, prompt, Wp1, bp1, Wp2, bp2)


def kernel(x, edge_index, edge_attr, We0, be0, W10, b10, W20, b20,
           We1, be1, W11, b11, W21, b21, We2, be2, W12, b12, W22, b22,
           gamma, beta, prompt, Wp1, bp1, Wp2, bp2):
    r2 = lambda v: v.reshape(1, -1)
    ep0, ep1, ep2 = _edge_proj(edge_attr.T, We0, r2(be0), We1, r2(be1),
                               We2, r2(be2))

    h = x
    agg = _sc_aggregate(h, ep0, edge_index)
    h = _node_mlp(h, agg, W10, r2(b10), W20, r2(b20))

    agg = _sc_aggregate(h, ep1, edge_index)
    h = _node_mlp(h, agg, W11, r2(b11), W21, r2(b21))

    agg = _sc_aggregate(h, ep2, edge_index)
    return _node_mlp_final(h, agg, W12, r2(b12), W22, r2(b22),
                           r2(gamma), r2(beta), prompt,
                           Wp1, r2(bp1), Wp2, r2(bp2))
